# baseline (speedup 1.0000x reference)
import numpy as np
import jax
import jax.numpy as jnp
from jax.experimental import pallas as pl
from jax.experimental.pallas import tpu as pltpu


def _resize_mat(out_size: int, in_size: int) -> np.ndarray:
    scale = out_size / in_size
    sample_f = (np.arange(out_size) + 0.5) / scale - 0.5
    x = np.abs(sample_f[:, None] - np.arange(in_size)[None, :])
    w = np.maximum(0.0, 1.0 - x)
    w = w / w.sum(axis=1, keepdims=True)
    return w.astype(np.float32)


def _upsample_mat() -> np.ndarray:
    u = _resize_mat(28, 14)
    p = np.zeros((784, 196), dtype=np.float32)
    for y in range(28):
        for x in range(28):
            p[y * 28 + x] = np.kron(u[y], u[x])
    return p


def _conv1_kernel(x_ref, lsel_ref, ls_ref, m_ref, wkt_ref, b_ref,
                  out_ref, q2a_ref):
    ls = ls_ref[...]
    msk = m_ref[...]
    acc = None
    for c in range(3):
        v = x_ref[0, c]
        for ky in range(8):
            s = c * 8 + ky
            slab = jnp.dot(lsel_ref[ky], v,
                           preferred_element_type=jnp.float32)
            t = jnp.dot(ls, slab, preferred_element_type=jnp.float32)
            d = jnp.dot(t * msk, wkt_ref[s],
                        preferred_element_type=jnp.float32)
            acc = d if acc is None else acc + d
    f2 = jnp.maximum(acc + b_ref[...], 0.0)
    out_ref[0] = f2
    q2a_ref[0] = jnp.sum(f2.reshape(28, 28, 128) ** 2, axis=2)


def _feat3_kernel(a0_ref, a1_ref, w0_ref, w1_ref, w2_ref,
                  w3_ref, b_ref, p_ref, q2a_ref, qb_ref, q2_ref):
    v0 = a0_ref[0, :, 0, :, :]
    v1 = a1_ref[0, :, 0, :, :]
    v0f, v1f = v0.reshape(196, 256), v1.reshape(196, 256)
    a0, a1 = v0f[:, :128], v0f[:, 128:]
    a2, a3 = v1f[:, :128], v1f[:, 128:]
    acc = jnp.dot(a0, w0_ref[...], preferred_element_type=jnp.float32)
    acc += jnp.dot(a1, w1_ref[...], preferred_element_type=jnp.float32)
    acc += jnp.dot(a2, w2_ref[...], preferred_element_type=jnp.float32)
    acc += jnp.dot(a3, w3_ref[...], preferred_element_type=jnp.float32)
    f3 = jnp.maximum(acc + b_ref[...], 0.0)
    f3u = jnp.dot(p_ref[...], f3, preferred_element_type=jnp.float32)
    qb_ref[0, :, 0, :, :128] = v0.reshape(14, 28, 128).astype(jnp.bfloat16)
    qb_ref[0, :, 1, :, :128] = v1.reshape(14, 28, 128).astype(jnp.bfloat16)
    qb_ref[0, :, :, :, 128:] = (
        f3u.reshape(14, 2, 28, 256).astype(jnp.bfloat16))
    q2_ref[0] = q2a_ref[0] + jnp.sum(f3u.reshape(28, 28, 256) ** 2, axis=2)


def _bank_prep_kernel(m_ref, mb_ref, mh_ref):
    m = m_ref[...]
    mb_ref[...] = m.astype(jnp.bfloat16)
    mh_ref[0, :] = 0.5 * jnp.sum(m * m, axis=1)


def _cdist_max_kernel(qb_ref, mb_ref, mh_ref, out_ref):
    j = pl.program_id(1)
    qb = qb_ref[...]
    nc, cs = 4, mb_ref.shape[0] // 4
    parts = []
    for k in range(nc):
        acc = jax.lax.dot_general(qb, mb_ref[k * cs:(k + 1) * cs, :],
                                  (((1,), (1,)), ((), ())),
                                  preferred_element_type=jnp.float32)
        parts.append(jnp.max(acc - mh_ref[:, k * cs:(k + 1) * cs], axis=1))
    rowmax = jnp.maximum(jnp.maximum(parts[0], parts[1]),
                         jnp.maximum(parts[2], parts[3]))[:, None]

    @pl.when(j == 0)
    def _init():
        out_ref[...] = rowmax

    @pl.when(j > 0)
    def _acc():
        out_ref[...] = jnp.maximum(out_ref[...], rowmax)


def _resize_max_kernel(q2_ref, mx_ref, a_ref, at_ref, map_ref, score_ref):
    d2 = q2_ref[0] - 2.0 * mx_ref[0]
    m = jnp.sqrt(jnp.maximum(d2, 0.0))
    t = jnp.dot(a_ref[...], m, preferred_element_type=jnp.float32)
    up = jnp.dot(t, at_ref[...], preferred_element_type=jnp.float32)
    map_ref[0] = up
    score_ref[0, 0, :] = jnp.full((128,), jnp.max(up), jnp.float32)


@jax.jit
def kernel(x, W1, b1, W2, b2, memory_bank):
    B = x.shape[0]
    NQ = B * 784
    TQ, TM = 448, 16384
    G1 = 8

    wk = W1.reshape(128, 3, 8, 8).transpose(1, 2, 3, 0)
    wk = wk.reshape(24, 8, 128)
    wkt = jnp.tile(wk, (1, 28, 1))
    lsel = np.zeros((8, 28, 224), dtype=np.float32)
    for ky in range(8):
        lsel[ky, np.arange(28), np.arange(28) * 8 + ky] = 1.0
    lsel = jnp.asarray(lsel)
    r_idx = np.arange(784)
    ls = np.zeros((784, 28), dtype=np.float32)
    ls[r_idx, r_idx // 28] = 1.0
    ls = jnp.asarray(ls)
    msk = (r_idx[:, None] % 28 == np.arange(224)[None, :] // 8)
    msk = jnp.asarray(msk.astype(np.float32))
    w2r = W2.transpose(2, 3, 1, 0).reshape(4, 128, 256)
    p_up = jnp.asarray(_upsample_mat())
    a28 = jnp.asarray(_resize_mat(224, 28))

    f2, q2a = pl.pallas_call(
        _conv1_kernel,
        grid=(B,),
        in_specs=[
            pl.BlockSpec((1, 3, 224, 224), lambda i: (i, 0, 0, 0)),
            pl.BlockSpec((8, 28, 224), lambda i: (0, 0, 0)),
            pl.BlockSpec((784, 28), lambda i: (0, 0)),
            pl.BlockSpec((784, 224), lambda i: (0, 0)),
            pl.BlockSpec((24, 224, 128), lambda i: (0, 0, 0)),
            pl.BlockSpec((1, 128), lambda i: (0, 0)),
        ],
        out_specs=[
            pl.BlockSpec((1, 784, 128), lambda i: (i, 0, 0)),
            pl.BlockSpec((1, 28, 28), lambda i: (i, 0, 0)),
        ],
        out_shape=[
            jax.ShapeDtypeStruct((B, 784, 128), jnp.float32),
            jax.ShapeDtypeStruct((B, 28, 28), jnp.float32),
        ],
    )(x, lsel, ls, msk, wkt, b1.reshape(1, 128))

    f2v = f2.reshape(B, 14, 2, 14, 256)
    qbp, q2 = pl.pallas_call(
        _feat3_kernel,
        grid=(B,),
        in_specs=[
            pl.BlockSpec((1, 14, 1, 14, 256),
                         lambda i, di=di: (i, 0, di, 0, 0))
            for di in range(2)
        ]
        + [pl.BlockSpec((128, 256), lambda i: (0, 0))] * 4
        + [
            pl.BlockSpec((1, 256), lambda i: (0, 0)),
            pl.BlockSpec((784, 196), lambda i: (0, 0)),
            pl.BlockSpec((1, 28, 28), lambda i: (i, 0, 0)),
        ],
        out_specs=[
            pl.BlockSpec((1, 14, 2, 28, 384), lambda i: (i, 0, 0, 0, 0)),
            pl.BlockSpec((1, 28, 28), lambda i: (i, 0, 0)),
        ],
        out_shape=[
            jax.ShapeDtypeStruct((B, 14, 2, 28, 384), jnp.bfloat16),
            jax.ShapeDtypeStruct((B, 28, 28), jnp.float32),
        ],
    )(f2v, f2v, w2r[0], w2r[1], w2r[2], w2r[3],
      b2.reshape(1, 256), p_up, q2a)

    qb = qbp.reshape(NQ, 384)

    nm = memory_bank.shape[0]
    mb, mh = pl.pallas_call(
        _bank_prep_kernel,
        grid=(nm // TM,),
        in_specs=[pl.BlockSpec((TM, 384), lambda j: (j, 0))],
        out_specs=[
            pl.BlockSpec((TM, 384), lambda j: (j, 0)),
            pl.BlockSpec((1, TM), lambda j: (0, j)),
        ],
        out_shape=[
            jax.ShapeDtypeStruct((nm, 384), jnp.bfloat16),
            jax.ShapeDtypeStruct((1, nm), jnp.float32),
        ],
    )(memory_bank)

    maxdot = pl.pallas_call(
        _cdist_max_kernel,
        grid=(NQ // TQ, nm // TM),
        in_specs=[
            pl.BlockSpec((TQ, 384), lambda i, j: (i, 0)),
            pl.BlockSpec((TM, 384), lambda i, j: (j, 0)),
            pl.BlockSpec((1, TM), lambda i, j: (0, j)),
        ],
        out_specs=pl.BlockSpec((TQ, 1), lambda i, j: (i, 0)),
        out_shape=jax.ShapeDtypeStruct((NQ, 1), jnp.float32),
        compiler_params=pltpu.CompilerParams(
            dimension_semantics=("parallel", "arbitrary")),
    )(qb, mb, mh)

    q2r = q2
    mxr = maxdot.reshape(B, 28, 28)

    amap, score = pl.pallas_call(
        _resize_max_kernel,
        grid=(B,),
        in_specs=[
            pl.BlockSpec((1, 28, 28), lambda i: (i, 0, 0)),
            pl.BlockSpec((1, 28, 28), lambda i: (i, 0, 0)),
            pl.BlockSpec((224, 28), lambda i: (0, 0)),
            pl.BlockSpec((28, 224), lambda i: (0, 0)),
        ],
        out_specs=[
            pl.BlockSpec((1, 224, 224), lambda i: (i, 0, 0)),
            pl.BlockSpec((1, 1, 128), lambda i: (i, 0, 0)),
        ],
        out_shape=[
            jax.ShapeDtypeStruct((B, 224, 224), jnp.float32),
            jax.ShapeDtypeStruct((B, 1, 128), jnp.float32),
        ],
    )(q2r, mxr, a28, a28.T)

    return amap.reshape(B, 1, 224, 224), score[:, 0, 0]


# speedup vs baseline: 1.0584x; 1.0584x over previous
import numpy as np
import jax
import jax.numpy as jnp
from jax.experimental import pallas as pl
from jax.experimental.pallas import tpu as pltpu


def _resize_mat(out_size: int, in_size: int) -> np.ndarray:
    scale = out_size / in_size
    sample_f = (np.arange(out_size) + 0.5) / scale - 0.5
    x = np.abs(sample_f[:, None] - np.arange(in_size)[None, :])
    w = np.maximum(0.0, 1.0 - x)
    w = w / w.sum(axis=1, keepdims=True)
    return w.astype(np.float32)


def _upsample_mat() -> np.ndarray:
    u = _resize_mat(28, 14)
    p = np.zeros((784, 196), dtype=np.float32)
    for y in range(28):
        for x in range(28):
            p[y * 28 + x] = np.kron(u[y], u[x])
    return p


def _features_kernel(x_ref, lsel_ref, ls_ref, m_ref, wkt_ref, b_ref,
                     w0_ref, w1_ref, w2_ref, w3_ref, b2_ref, p_ref,
                     qb_ref, q2_ref):
    ls = ls_ref[...]
    msk = m_ref[...]
    acc = None
    for c in range(3):
        v = x_ref[0, c]
        for ky in range(8):
            s = c * 8 + ky
            slab = jnp.dot(lsel_ref[ky], v,
                           preferred_element_type=jnp.float32)
            t = jnp.dot(ls, slab, preferred_element_type=jnp.float32)
            d = jnp.dot(t * msk, wkt_ref[s],
                        preferred_element_type=jnp.float32)
            acc = d if acc is None else acc + d
    f2 = jnp.maximum(acc + b_ref[...], 0.0)
    q2a = jnp.sum(f2.reshape(28, 28, 128) ** 2, axis=2)
    r5 = f2.reshape(14, 2, 14, 2, 128)
    a0 = r5[:, 0, :, 0, :].reshape(196, 128)
    a1 = r5[:, 0, :, 1, :].reshape(196, 128)
    a2 = r5[:, 1, :, 0, :].reshape(196, 128)
    a3 = r5[:, 1, :, 1, :].reshape(196, 128)
    acc2 = jnp.dot(a0, w0_ref[...], preferred_element_type=jnp.float32)
    acc2 += jnp.dot(a1, w1_ref[...], preferred_element_type=jnp.float32)
    acc2 += jnp.dot(a2, w2_ref[...], preferred_element_type=jnp.float32)
    acc2 += jnp.dot(a3, w3_ref[...], preferred_element_type=jnp.float32)
    f3 = jnp.maximum(acc2 + b2_ref[...], 0.0)
    f3u = jnp.dot(p_ref[...], f3, preferred_element_type=jnp.float32)
    f2r = f2.reshape(14, 2, 28, 128)
    qb_ref[0, :, 0, :, :128] = f2r[:, 0].astype(jnp.bfloat16)
    qb_ref[0, :, 1, :, :128] = f2r[:, 1].astype(jnp.bfloat16)
    qb_ref[0, :, :, :, 128:] = (
        f3u.reshape(14, 2, 28, 256).astype(jnp.bfloat16))
    q2_ref[0] = q2a + jnp.sum(f3u.reshape(28, 28, 256) ** 2, axis=2)


def _bank_prep_kernel(m_ref, mb_ref, mh_ref):
    m = m_ref[...]
    mb_ref[...] = m.astype(jnp.bfloat16)
    mh_ref[0, :] = 0.5 * jnp.sum(m * m, axis=1)


def _cdist_max_kernel(qb_ref, mb_ref, mh_ref, out_ref):
    j = pl.program_id(1)
    qb = qb_ref[...]
    nc, cs = 4, mb_ref.shape[0] // 4
    parts = []
    for k in range(nc):
        acc = jax.lax.dot_general(qb, mb_ref[k * cs:(k + 1) * cs, :],
                                  (((1,), (1,)), ((), ())),
                                  preferred_element_type=jnp.float32)
        parts.append(jnp.max(acc - mh_ref[:, k * cs:(k + 1) * cs], axis=1))
    rowmax = jnp.maximum(jnp.maximum(parts[0], parts[1]),
                         jnp.maximum(parts[2], parts[3]))[:, None]

    @pl.when(j == 0)
    def _init():
        out_ref[...] = rowmax

    @pl.when(j > 0)
    def _acc():
        out_ref[...] = jnp.maximum(out_ref[...], rowmax)


def _resize_max_kernel(q2_ref, mx_ref, a_ref, at_ref, map_ref, score_ref):
    d2 = q2_ref[0] - 2.0 * mx_ref[0]
    m = jnp.sqrt(jnp.maximum(d2, 0.0))
    t = jnp.dot(a_ref[...], m, preferred_element_type=jnp.float32)
    up = jnp.dot(t, at_ref[...], preferred_element_type=jnp.float32)
    map_ref[0] = up
    score_ref[0, 0, :] = jnp.full((128,), jnp.max(up), jnp.float32)


@jax.jit
def kernel(x, W1, b1, W2, b2, memory_bank):
    B = x.shape[0]
    NQ = B * 784
    TQ, TM = 896, 8192
    G1 = 8

    wk = W1.reshape(128, 3, 8, 8).transpose(1, 2, 3, 0)
    wk = wk.reshape(24, 8, 128)
    wkt = jnp.tile(wk, (1, 28, 1))
    lsel = np.zeros((8, 28, 224), dtype=np.float32)
    for ky in range(8):
        lsel[ky, np.arange(28), np.arange(28) * 8 + ky] = 1.0
    lsel = jnp.asarray(lsel)
    r_idx = np.arange(784)
    ls = np.zeros((784, 28), dtype=np.float32)
    ls[r_idx, r_idx // 28] = 1.0
    ls = jnp.asarray(ls)
    msk = (r_idx[:, None] % 28 == np.arange(224)[None, :] // 8)
    msk = jnp.asarray(msk.astype(np.float32))
    w2r = W2.transpose(2, 3, 1, 0).reshape(4, 128, 256)
    p_up = jnp.asarray(_upsample_mat())
    a28 = jnp.asarray(_resize_mat(224, 28))

    qbp, q2 = pl.pallas_call(
        _features_kernel,
        grid=(B,),
        in_specs=[
            pl.BlockSpec((1, 3, 224, 224), lambda i: (i, 0, 0, 0)),
            pl.BlockSpec((8, 28, 224), lambda i: (0, 0, 0)),
            pl.BlockSpec((784, 28), lambda i: (0, 0)),
            pl.BlockSpec((784, 224), lambda i: (0, 0)),
            pl.BlockSpec((24, 224, 128), lambda i: (0, 0, 0)),
            pl.BlockSpec((1, 128), lambda i: (0, 0)),
        ]
        + [pl.BlockSpec((128, 256), lambda i: (0, 0))] * 4
        + [
            pl.BlockSpec((1, 256), lambda i: (0, 0)),
            pl.BlockSpec((784, 196), lambda i: (0, 0)),
        ],
        out_specs=[
            pl.BlockSpec((1, 14, 2, 28, 384), lambda i: (i, 0, 0, 0, 0)),
            pl.BlockSpec((1, 28, 28), lambda i: (i, 0, 0)),
        ],
        out_shape=[
            jax.ShapeDtypeStruct((B, 14, 2, 28, 384), jnp.bfloat16),
            jax.ShapeDtypeStruct((B, 28, 28), jnp.float32),
        ],
    )(x, lsel, ls, msk, wkt, b1.reshape(1, 128),
      w2r[0], w2r[1], w2r[2], w2r[3], b2.reshape(1, 256), p_up)

    qb = qbp.reshape(NQ, 384)

    nm = memory_bank.shape[0]
    mb, mh = pl.pallas_call(
        _bank_prep_kernel,
        grid=(nm // TM,),
        in_specs=[pl.BlockSpec((TM, 384), lambda j: (j, 0))],
        out_specs=[
            pl.BlockSpec((TM, 384), lambda j: (j, 0)),
            pl.BlockSpec((1, TM), lambda j: (0, j)),
        ],
        out_shape=[
            jax.ShapeDtypeStruct((nm, 384), jnp.bfloat16),
            jax.ShapeDtypeStruct((1, nm), jnp.float32),
        ],
    )(memory_bank)

    maxdot = pl.pallas_call(
        _cdist_max_kernel,
        grid=(NQ // TQ, nm // TM),
        in_specs=[
            pl.BlockSpec((TQ, 384), lambda i, j: (i, 0)),
            pl.BlockSpec((TM, 384), lambda i, j: (j, 0)),
            pl.BlockSpec((1, TM), lambda i, j: (0, j)),
        ],
        out_specs=pl.BlockSpec((TQ, 1), lambda i, j: (i, 0)),
        out_shape=jax.ShapeDtypeStruct((NQ, 1), jnp.float32),
        compiler_params=pltpu.CompilerParams(
            dimension_semantics=("parallel", "arbitrary")),
    )(qb, mb, mh)

    q2r = q2
    mxr = maxdot.reshape(B, 28, 28)

    amap, score = pl.pallas_call(
        _resize_max_kernel,
        grid=(B,),
        in_specs=[
            pl.BlockSpec((1, 28, 28), lambda i: (i, 0, 0)),
            pl.BlockSpec((1, 28, 28), lambda i: (i, 0, 0)),
            pl.BlockSpec((224, 28), lambda i: (0, 0)),
            pl.BlockSpec((28, 224), lambda i: (0, 0)),
        ],
        out_specs=[
            pl.BlockSpec((1, 224, 224), lambda i: (i, 0, 0)),
            pl.BlockSpec((1, 1, 128), lambda i: (i, 0, 0)),
        ],
        out_shape=[
            jax.ShapeDtypeStruct((B, 224, 224), jnp.float32),
            jax.ShapeDtypeStruct((B, 1, 128), jnp.float32),
        ],
    )(q2r, mxr, a28, a28.T)

    return amap.reshape(B, 1, 224, 224), score[:, 0, 0]
